# Initial kernel scaffold; baseline (speedup 1.0000x reference)
import jax
import jax.numpy as jnp
from jax.experimental import pallas as pl


def kernel(pred, target):
    raise NotImplementedError("write your pallas kernel here")


# speedup vs baseline: 25.3326x; 25.3326x over previous
import functools

import jax
import jax.numpy as jnp
import numpy as np
from jax.experimental import pallas as pl
from jax.experimental.pallas import tpu as pltpu

_ROWS = 16384
_COLS = 1024
_BINS = 10
_BLOCK_ROWS = 1024


def _body(pred_ref, target_ref, out_ref, acc_ref):
    i = pl.program_id(0)
    nsteps = pl.num_programs(0)

    @pl.when(i == 0)
    def _init():
        for r in range(2):
            for b in range(_BINS):
                acc_ref[r, b] = jnp.float32(0.0)

    p = pred_ref[...]
    t = target_ref[...]
    g = jnp.abs(jax.nn.sigmoid(p) - t)
    loss = jnp.maximum(p, 0.0) - p * t + jnp.log1p(jnp.exp(-jnp.abs(p)))

    for b in range(_BINS - 1):
        thr = np.float32((b + 1) / _BINS)
        mask = g < thr
        acc_ref[0, b] += jnp.sum(mask.astype(jnp.float32))
        acc_ref[1, b] += jnp.sum(jnp.where(mask, loss, 0.0))
    acc_ref[1, _BINS - 1] += jnp.sum(loss)

    @pl.when(i == nsteps - 1)
    def _fin():
        tot = np.float32(_ROWS * _COLS)
        n = jnp.float32(0.0)
        acc = jnp.float32(0.0)
        prev_c = jnp.float32(0.0)
        prev_s = jnp.float32(0.0)
        for b in range(_BINS):
            c_cum = acc_ref[0, b] if b < _BINS - 1 else tot
            s_cum = acc_ref[1, b]
            c_b = c_cum - prev_c
            s_b = s_cum - prev_s
            nonempty = c_b > 0.0
            n += jnp.where(nonempty, 1.0, 0.0)
            acc += jnp.where(nonempty, s_b / jnp.where(nonempty, c_b, 1.0), 0.0)
            prev_c = c_cum
            prev_s = s_cum
        out_ref[0, 0] = acc / n


@functools.partial(jax.jit)
def kernel(pred, target):
    grid = (_ROWS // _BLOCK_ROWS,)
    out = pl.pallas_call(
        _body,
        grid=grid,
        in_specs=[
            pl.BlockSpec((_BLOCK_ROWS, _COLS), lambda i: (i, 0)),
            pl.BlockSpec((_BLOCK_ROWS, _COLS), lambda i: (i, 0)),
        ],
        out_specs=pl.BlockSpec(memory_space=pltpu.SMEM),
        out_shape=jax.ShapeDtypeStruct((1, 1), jnp.float32),
        scratch_shapes=[pltpu.SMEM((2, _BINS), jnp.float32)],
        compiler_params=pltpu.CompilerParams(
            dimension_semantics=("arbitrary",),
        ),
    )(pred, target)
    return out[0, 0]
